# baseline (speedup 1.0000x reference)
import jax
import jax.numpy as jnp
from jax.experimental import pallas as pl

_B = 4096
_NN = 4
_D = 128
_R = 16
_NB = 10
_EPG = 6
_GB = 128
_NBLK = _B // _GB
_EB = _GB * _EPG
_NBL = _GB * _NN
_NQ = _NN * _R
_NC = _NN * _NB


def _roll_nodes(a3, d):
    return jnp.concatenate([a3[:, _NN - d:, :], a3[:, :_NN - d, :]],
                           axis=1).reshape(_NBL, _D)


def _fused(src_c_ref, dst_c_ref, et_c_ref, dst_r_ref, x_ref,
           t_ref, bstack_ref, comp_ref, root_ref, bias_ref, exp_ref,
           out_ref):
    blk = pl.program_id(0)

    src_c = src_c_ref[...]
    dst_c = dst_c_ref[...]
    et_c = et_c_ref[...]
    eloc_c = jax.lax.broadcasted_iota(jnp.int32, (_EB, 1), 0)
    gloc_c = eloc_c // _EPG
    base_c = (blk * _GB + gloc_c) * _NN
    srcl_c = src_c - base_c
    dstl_c = dst_c - base_c

    dst_r = dst_r_ref[...].reshape(1, _EB)

    delta_c = (dstl_c - srcl_c + _NN) & (_NN - 1)
    code_c = delta_c * _R + et_c
    ow1 = (code_c == jax.lax.broadcasted_iota(jnp.int32, (1, _NQ), 1)
           ).astype(jnp.bfloat16)
    nid = jax.lax.broadcasted_iota(jnp.int32, (_NBL, 1), 0)
    g2 = (nid == (dst_r - blk * _NBL)).astype(jnp.bfloat16)
    s2raw = jnp.dot(g2, ow1, preferred_element_type=jnp.float32)

    fold = ((jax.lax.broadcasted_iota(jnp.int32, (_NQ, _R), 0) % _R) ==
            jax.lax.broadcasted_iota(jnp.int32, (_NQ, _R), 1)
            ).astype(jnp.float32)
    cnt16 = jnp.dot(s2raw, fold, preferred_element_type=jnp.float32)
    rc16 = 1.0 / jnp.maximum(cnt16, 1.0)
    rc64 = jnp.dot(rc16, fold.T, preferred_element_type=jnp.float32)
    s2 = s2raw * rc64

    comp = comp_ref[...]
    bsel = ((jax.lax.broadcasted_iota(jnp.int32, (_NB, _NC), 1) % _NB) ==
            jax.lax.broadcasted_iota(jnp.int32, (_NB, _NC), 0)
            ).astype(jnp.float32)
    dmask = ((jax.lax.broadcasted_iota(jnp.int32, (_NQ, _NC), 0) // _R) ==
             (jax.lax.broadcasted_iota(jnp.int32, (_NQ, _NC), 1) // _NB)
             ).astype(jnp.float32)
    k3 = jnp.dot(fold, jnp.dot(comp, bsel,
                               preferred_element_type=jnp.float32),
                 preferred_element_type=jnp.float32) * dmask

    cd_all = jnp.dot(s2, k3, preferred_element_type=jnp.float32)
    cdexp = jnp.dot(cd_all.astype(jnp.bfloat16), exp_ref[...],
                    preferred_element_type=jnp.float32)

    x = x_ref[...]
    x3 = x.reshape(_GB, _NN, _D)
    xsh = [x] + [_roll_nodes(x3, d) for d in range(1, _NN)]

    bstack = bstack_ref[...]
    rootm = root_ref[...]
    bvec = bias_ref[...]

    parts = []
    for b in range(_NB):
        acc = cdexp[:, b * _D:(b + 1) * _D] * xsh[0]
        for d in range(1, _NN):
            c = d * _NB + b
            acc = acc + cdexp[:, c * _D:(c + 1) * _D] * xsh[d]
        parts.append(acc)
    ycat = jnp.concatenate(parts, axis=1)
    agg = jnp.dot(ycat, bstack, preferred_element_type=jnp.float32)
    agg = agg + jnp.dot(x, rootm, preferred_element_type=jnp.float32) + bvec
    h = jnp.maximum(agg, 0.0)

    s23 = s2.reshape(_GB, _NN, _NQ)
    wsin = jnp.concatenate(
        [s2[:, :_R]] +
        [jnp.concatenate([s23[:, d:, :], s23[:, :d, :]], axis=1
                         ).reshape(_NBL, _NQ)[:, d * _R:(d + 1) * _R]
         for d in range(1, _NN)], axis=1)
    ws = jnp.dot(wsin, jnp.concatenate([comp] * _NN, axis=0),
                 preferred_element_type=jnp.float32)
    wexp = jnp.dot(ws.astype(jnp.bfloat16), exp_ref[:_NB, :_NB * _D],
                   preferred_element_type=jnp.float32)
    uparts = [wexp[:, b * _D:(b + 1) * _D] * h for b in range(_NB)]
    u = jnp.concatenate(uparts, axis=1)
    rg = (jax.lax.broadcasted_iota(jnp.int32, (_GB, _NBL), 0) ==
          jax.lax.broadcasted_iota(jnp.int32, (_GB, _NBL), 1) // _NN
          ).astype(jnp.float32)
    zcat = jnp.dot(rg, u, preferred_element_type=jnp.float32)
    hsum = jnp.dot(rg, h, preferred_element_type=jnp.float32)
    gvec = jnp.dot(zcat, bstack, preferred_element_type=jnp.float32)
    gvec = gvec + jnp.dot(hsum, rootm,
                          preferred_element_type=jnp.float32) + _NN * bvec

    t = t_ref[...]
    num = jnp.sum(gvec * t, axis=1)
    den = jnp.sqrt(jnp.sum(gvec * gvec, axis=1)) * jnp.sqrt(jnp.sum(t * t,
                                                                    axis=1))
    out_ref[0, 0, :] = num / jnp.maximum(den, 1e-8)


def kernel(x, edge_index, edge_type, batch_idx, target_embeds, bases, comp,
           root, bias):
    src_c = edge_index[0].reshape(_B * _EPG, 1)
    dst_c = edge_index[1].reshape(_B * _EPG, 1)
    et_c = edge_type.reshape(_B * _EPG, 1)
    dst_r = edge_index[1].reshape(_NBLK, 1, _EB)
    bstack = bases.reshape(_NB * _D, _D)
    bias2 = bias.reshape(1, _D)
    expand = jnp.repeat(jnp.eye(_NC, dtype=jnp.bfloat16), _D, axis=1)
    out = pl.pallas_call(
        _fused,
        grid=(_NBLK,),
        in_specs=[
            pl.BlockSpec((_EB, 1), lambda i: (i, 0)),
            pl.BlockSpec((_EB, 1), lambda i: (i, 0)),
            pl.BlockSpec((_EB, 1), lambda i: (i, 0)),
            pl.BlockSpec((1, 1, _EB), lambda i: (i, 0, 0)),
            pl.BlockSpec((_NBL, _D), lambda i: (i, 0)),
            pl.BlockSpec((_GB, _D), lambda i: (i, 0)),
            pl.BlockSpec((_NB * _D, _D), lambda i: (0, 0)),
            pl.BlockSpec((_R, _NB), lambda i: (0, 0)),
            pl.BlockSpec((_D, _D), lambda i: (0, 0)),
            pl.BlockSpec((1, _D), lambda i: (0, 0)),
            pl.BlockSpec((_NC, _NC * _D), lambda i: (0, 0)),
        ],
        out_specs=pl.BlockSpec((1, 1, _GB), lambda i: (i, 0, 0)),
        out_shape=jax.ShapeDtypeStruct((_NBLK, 1, _GB), jnp.float32),
    )(src_c, dst_c, et_c, dst_r, x, target_embeds, bstack, comp,
      root, bias2, expand)
    return out.reshape(_B)


# speedup vs baseline: 1.0121x; 1.0121x over previous
import jax
import jax.numpy as jnp
from jax.experimental import pallas as pl

_B = 4096
_NN = 4
_D = 128
_R = 16
_NB = 10
_EPG = 6
_GB = 128
_NBLK = _B // _GB
_EB = _GB * _EPG
_NBL = _GB * _NN
_NQ = _NN * _R
_NC = _NN * _NB


def _roll_nodes(a3, d):
    return jnp.concatenate([a3[:, _NN - d:, :], a3[:, :_NN - d, :]],
                           axis=1).reshape(_NBL, _D)


def _fused(src_c_ref, dst_c_ref, et_c_ref, dst_r_ref, x_ref,
           t_ref, bstack_ref, comp_ref, root_ref, bias_ref, exp_ref,
           out_ref):
    blk = pl.program_id(0)

    src_c = src_c_ref[...]
    dst_c = dst_c_ref[...]
    et_c = et_c_ref[...]
    eloc_c = jax.lax.broadcasted_iota(jnp.int32, (_EB, 1), 0)
    gloc_c = eloc_c // _EPG
    base_c = (blk * _GB + gloc_c) * _NN
    srcl_c = src_c - base_c
    dstl_c = dst_c - base_c

    dst_r = dst_r_ref[...].reshape(1, _EB)

    delta_c = (dstl_c - srcl_c + _NN) & (_NN - 1)
    code_c = delta_c * _R + et_c
    ow1 = (code_c == jax.lax.broadcasted_iota(jnp.int32, (1, _NQ), 1)
           ).astype(jnp.bfloat16)
    nid = jax.lax.broadcasted_iota(jnp.int32, (_NBL, 1), 0)
    g2 = (nid == (dst_r - blk * _NBL)).astype(jnp.bfloat16)
    s2raw = jnp.dot(g2, ow1, preferred_element_type=jnp.float32)

    fold = ((jax.lax.broadcasted_iota(jnp.int32, (_NQ, _R), 0) % _R) ==
            jax.lax.broadcasted_iota(jnp.int32, (_NQ, _R), 1)
            ).astype(jnp.float32)
    cnt16 = jnp.dot(s2raw, fold, preferred_element_type=jnp.float32)
    rc16 = 1.0 / jnp.maximum(cnt16, 1.0)
    rc64 = jnp.dot(rc16, fold.T, preferred_element_type=jnp.float32)
    s2 = s2raw * rc64

    comp = comp_ref[...]
    bsel = ((jax.lax.broadcasted_iota(jnp.int32, (_NB, _NC), 1) % _NB) ==
            jax.lax.broadcasted_iota(jnp.int32, (_NB, _NC), 0)
            ).astype(jnp.float32)
    dmask = ((jax.lax.broadcasted_iota(jnp.int32, (_NQ, _NC), 0) // _R) ==
             (jax.lax.broadcasted_iota(jnp.int32, (_NQ, _NC), 1) // _NB)
             ).astype(jnp.float32)
    k3 = jnp.dot(fold, jnp.dot(comp, bsel,
                               preferred_element_type=jnp.float32),
                 preferred_element_type=jnp.float32) * dmask

    cd_all = jnp.dot(s2, k3, preferred_element_type=jnp.float32)
    cdexp = jnp.dot(cd_all, exp_ref[...],
                    preferred_element_type=jnp.float32)

    x = x_ref[...]
    x3 = x.reshape(_GB, _NN, _D)
    xsh = [x] + [_roll_nodes(x3, d) for d in range(1, _NN)]

    bstack = bstack_ref[...]
    rootm = root_ref[...]
    bvec = bias_ref[...]

    parts = []
    for b in range(_NB):
        acc = cdexp[:, b * _D:(b + 1) * _D] * xsh[0]
        for d in range(1, _NN):
            c = d * _NB + b
            acc = acc + cdexp[:, c * _D:(c + 1) * _D] * xsh[d]
        parts.append(acc)
    ycat = jnp.concatenate(parts, axis=1)
    agg = jnp.dot(ycat, bstack, preferred_element_type=jnp.float32)
    agg = agg + jnp.dot(x, rootm, preferred_element_type=jnp.float32) + bvec
    h = jnp.maximum(agg, 0.0)

    s23 = s2.reshape(_GB, _NN, _NQ)
    wsin = jnp.concatenate(
        [s2[:, :_R]] +
        [jnp.concatenate([s23[:, d:, :], s23[:, :d, :]], axis=1
                         ).reshape(_NBL, _NQ)[:, d * _R:(d + 1) * _R]
         for d in range(1, _NN)], axis=1)
    ws = jnp.dot(wsin, jnp.concatenate([comp] * _NN, axis=0),
                 preferred_element_type=jnp.float32)
    wexp = jnp.dot(ws, exp_ref[:_NB, :_NB * _D],
                   preferred_element_type=jnp.float32)
    uparts = [wexp[:, b * _D:(b + 1) * _D] * h for b in range(_NB)]
    u = jnp.concatenate(uparts, axis=1)
    rg = (jax.lax.broadcasted_iota(jnp.int32, (_GB, _NBL), 0) ==
          jax.lax.broadcasted_iota(jnp.int32, (_GB, _NBL), 1) // _NN
          ).astype(jnp.float32)
    zcat = jnp.dot(rg, u, preferred_element_type=jnp.float32)
    hsum = jnp.dot(rg, h, preferred_element_type=jnp.float32)
    gvec = jnp.dot(zcat, bstack, preferred_element_type=jnp.float32)
    gvec = gvec + jnp.dot(hsum, rootm,
                          preferred_element_type=jnp.float32) + _NN * bvec

    t = t_ref[...]
    num = jnp.sum(gvec * t, axis=1)
    den = jnp.sqrt(jnp.sum(gvec * gvec, axis=1)) * jnp.sqrt(jnp.sum(t * t,
                                                                    axis=1))
    out_ref[0, 0, :] = num / jnp.maximum(den, 1e-8)


def kernel(x, edge_index, edge_type, batch_idx, target_embeds, bases, comp,
           root, bias):
    src_c = edge_index[0].reshape(_B * _EPG, 1)
    dst_c = edge_index[1].reshape(_B * _EPG, 1)
    et_c = edge_type.reshape(_B * _EPG, 1)
    dst_r = edge_index[1].reshape(_NBLK, 1, _EB)
    bstack = bases.reshape(_NB * _D, _D)
    bias2 = bias.reshape(1, _D)
    expand = jnp.repeat(jnp.eye(_NC, dtype=jnp.float32), _D, axis=1)
    out = pl.pallas_call(
        _fused,
        grid=(_NBLK,),
        in_specs=[
            pl.BlockSpec((_EB, 1), lambda i: (i, 0)),
            pl.BlockSpec((_EB, 1), lambda i: (i, 0)),
            pl.BlockSpec((_EB, 1), lambda i: (i, 0)),
            pl.BlockSpec((1, 1, _EB), lambda i: (i, 0, 0)),
            pl.BlockSpec((_NBL, _D), lambda i: (i, 0)),
            pl.BlockSpec((_GB, _D), lambda i: (i, 0)),
            pl.BlockSpec((_NB * _D, _D), lambda i: (0, 0)),
            pl.BlockSpec((_R, _NB), lambda i: (0, 0)),
            pl.BlockSpec((_D, _D), lambda i: (0, 0)),
            pl.BlockSpec((1, _D), lambda i: (0, 0)),
            pl.BlockSpec((_NC, _NC * _D), lambda i: (0, 0)),
        ],
        out_specs=pl.BlockSpec((1, 1, _GB), lambda i: (i, 0, 0)),
        out_shape=jax.ShapeDtypeStruct((_NBLK, 1, _GB), jnp.float32),
    )(src_c, dst_c, et_c, dst_r, x, target_embeds, bstack, comp,
      root, bias2, expand)
    return out.reshape(_B)


# speedup vs baseline: 1.0653x; 1.0526x over previous
import functools

import jax
import jax.numpy as jnp
from jax import lax
from jax.experimental import pallas as pl
from jax.experimental.pallas import tpu as pltpu
from jax.experimental.pallas import tpu_sc as plsc

_B = 4096
_NN = 4
_D = 128
_R = 16
_NB = 10
_EPG = 6
_GB = 128
_NBLK = _B // _GB
_NBL = _GB * _NN
_NC = _NN * _NB

_NWORK = 32
_GW = _B // _NWORK
_EW = _GW * _EPG
_NWN = _GW * _NN
_CNTN = _NWN * _R
_CDN = _NWN * _NC
_WSN = _NWN * _NB


def _sc_mix_body(src_hbm, dst_hbm, et_hbm, comp_hbm, cd_hbm, ws_hbm,
                 src_v, dst_v, et_v, comp_v, cnt_v, cd_v, ws_v):
    c = lax.axis_index("c")
    s = lax.axis_index("s")
    wid = s * 2 + c
    ebase = wid * _EW
    nbase = wid * _NWN

    pltpu.sync_copy(src_hbm.at[pl.ds(ebase, _EW)], src_v)
    pltpu.sync_copy(dst_hbm.at[pl.ds(ebase, _EW)], dst_v)
    pltpu.sync_copy(et_hbm.at[pl.ds(ebase, _EW)], et_v)
    pltpu.sync_copy(comp_hbm, comp_v)

    zf = jnp.zeros((16,), jnp.float32)

    def _zero(ref, n):
        def body(i, carry):
            ref[pl.ds(i * 16, 16)] = zf
            return carry
        lax.fori_loop(0, n // 16, body, 0)

    _zero(cnt_v, _CNTN)
    _zero(cd_v, _CDN)
    _zero(ws_v, _WSN)

    lane = lax.broadcasted_iota(jnp.int32, (16,), 0)
    ones = jnp.ones((16,), jnp.float32)

    def pass1(grp, carry):
        for j in range(_EPG):
            eidx = grp * (16 * _EPG) + lane * _EPG + j
            d = plsc.load_gather(dst_v, [eidx]) - nbase
            et = plsc.load_gather(et_v, [eidx])
            plsc.addupdate_scatter(cnt_v, [d * _R + et], ones)
        return carry

    lax.fori_loop(0, _GW // 16, pass1, 0)

    def pass2(grp, carry):
        for j in range(_EPG):
            eidx = grp * (16 * _EPG) + lane * _EPG + j
            sl = plsc.load_gather(src_v, [eidx]) - nbase
            d = plsc.load_gather(dst_v, [eidx]) - nbase
            et = plsc.load_gather(et_v, [eidx])
            delta = (d - sl) & (_NN - 1)
            cnt = plsc.load_gather(cnt_v, [d * _R + et])
            inv = 1.0 / cnt
            cdi = d * _NC + delta * _NB
            wsi = sl * _NB
            for b in range(_NB):
                w = plsc.load_gather(comp_v, [et * _NB + b]) * inv
                plsc.addupdate_scatter(cd_v, [cdi + b], w)
                plsc.addupdate_scatter(ws_v, [wsi + b], w)
        return carry

    lax.fori_loop(0, _GW // 16, pass2, 0)

    pltpu.sync_copy(cd_v, cd_hbm.at[pl.ds(wid * _CDN, _CDN)])
    pltpu.sync_copy(ws_v, ws_hbm.at[pl.ds(wid * _WSN, _WSN)])


_sc_mix = functools.partial(
    pl.kernel,
    out_type=[jax.ShapeDtypeStruct((_B * _NN * _NC,), jnp.float32),
              jax.ShapeDtypeStruct((_B * _NN * _NB,), jnp.float32)],
    mesh=plsc.VectorSubcoreMesh(core_axis_name="c", subcore_axis_name="s"),
    compiler_params=pltpu.CompilerParams(needs_layout_passes=False),
    scratch_types=[
        pltpu.VMEM((_EW,), jnp.int32),
        pltpu.VMEM((_EW,), jnp.int32),
        pltpu.VMEM((_EW,), jnp.int32),
        pltpu.VMEM((_R * _NB,), jnp.float32),
        pltpu.VMEM((_CNTN,), jnp.float32),
        pltpu.VMEM((_CDN,), jnp.float32),
        pltpu.VMEM((_WSN,), jnp.float32),
    ],
)(_sc_mix_body)


def _roll_nodes(a3, d):
    return jnp.concatenate([a3[:, _NN - d:, :], a3[:, :_NN - d, :]],
                           axis=1).reshape(_NBL, _D)


def _fused(cd_ref, ws_ref, x_ref, t_ref, bstack_ref, root_ref, bias_ref,
           exp_ref, out_ref):
    cdexp = jnp.dot(cd_ref[...], exp_ref[...],
                    preferred_element_type=jnp.float32)

    x = x_ref[...]
    x3 = x.reshape(_GB, _NN, _D)
    xsh = [x] + [_roll_nodes(x3, d) for d in range(1, _NN)]

    bstack = bstack_ref[...]
    rootm = root_ref[...]
    bvec = bias_ref[...]

    parts = []
    for b in range(_NB):
        acc = cdexp[:, b * _D:(b + 1) * _D] * xsh[0]
        for d in range(1, _NN):
            c = d * _NB + b
            acc = acc + cdexp[:, c * _D:(c + 1) * _D] * xsh[d]
        parts.append(acc)
    ycat = jnp.concatenate(parts, axis=1)
    agg = jnp.dot(ycat, bstack, preferred_element_type=jnp.float32)
    agg = agg + jnp.dot(x, rootm, preferred_element_type=jnp.float32) + bvec
    h = jnp.maximum(agg, 0.0)

    wexp = jnp.dot(ws_ref[...], exp_ref[:_NB, :_NB * _D],
                   preferred_element_type=jnp.float32)
    uparts = [wexp[:, b * _D:(b + 1) * _D] * h for b in range(_NB)]
    u = jnp.concatenate(uparts, axis=1)
    rg = (jax.lax.broadcasted_iota(jnp.int32, (_GB, _NBL), 0) ==
          jax.lax.broadcasted_iota(jnp.int32, (_GB, _NBL), 1) // _NN
          ).astype(jnp.float32)
    zcat = jnp.dot(rg, u, preferred_element_type=jnp.float32)
    hsum = jnp.dot(rg, h, preferred_element_type=jnp.float32)
    gvec = jnp.dot(zcat, bstack, preferred_element_type=jnp.float32)
    gvec = gvec + jnp.dot(hsum, rootm,
                          preferred_element_type=jnp.float32) + _NN * bvec

    t = t_ref[...]
    num = jnp.sum(gvec * t, axis=1)
    den = jnp.sqrt(jnp.sum(gvec * gvec, axis=1)) * jnp.sqrt(jnp.sum(t * t,
                                                                    axis=1))
    out_ref[0, 0, :] = num / jnp.maximum(den, 1e-8)


def kernel(x, edge_index, edge_type, batch_idx, target_embeds, bases, comp,
           root, bias):
    src = edge_index[0].reshape(_B * _EPG).astype(jnp.int32)
    dst = edge_index[1].reshape(_B * _EPG).astype(jnp.int32)
    et = edge_type.reshape(_B * _EPG).astype(jnp.int32)
    compf = comp.reshape(_R * _NB).astype(jnp.float32)

    cd_flat, ws_flat = _sc_mix(src, dst, et, compf)
    cd = cd_flat.reshape(_B * _NN, _NC)
    ws = ws_flat.reshape(_B * _NN, _NB)

    bstack = bases.reshape(_NB * _D, _D)
    bias2 = bias.reshape(1, _D)
    expand = jnp.repeat(jnp.eye(_NC, dtype=jnp.float32), _D, axis=1)
    out = pl.pallas_call(
        _fused,
        grid=(_NBLK,),
        in_specs=[
            pl.BlockSpec((_NBL, _NC), lambda i: (i, 0)),
            pl.BlockSpec((_NBL, _NB), lambda i: (i, 0)),
            pl.BlockSpec((_NBL, _D), lambda i: (i, 0)),
            pl.BlockSpec((_GB, _D), lambda i: (i, 0)),
            pl.BlockSpec((_NB * _D, _D), lambda i: (0, 0)),
            pl.BlockSpec((_D, _D), lambda i: (0, 0)),
            pl.BlockSpec((1, _D), lambda i: (0, 0)),
            pl.BlockSpec((_NC, _NC * _D), lambda i: (0, 0)),
        ],
        out_specs=pl.BlockSpec((1, 1, _GB), lambda i: (i, 0, 0)),
        out_shape=jax.ShapeDtypeStruct((_NBLK, 1, _GB), jnp.float32),
    )(cd, ws, x, target_embeds, bstack, root, bias2, expand)
    return out.reshape(_B)
